# Initial kernel scaffold; baseline (speedup 1.0000x reference)
import jax
import jax.numpy as jnp
from jax.experimental import pallas as pl


def kernel(x, Wq, bq, Wk, bk, Wv, bv, Wo, bo):
    raise NotImplementedError("write your pallas kernel here")


# speedup vs baseline: 9.7977x; 9.7977x over previous
import functools

import jax
import jax.numpy as jnp
from jax import lax
from jax.experimental import pallas as pl
from jax.experimental.pallas import tpu as pltpu
from jax.experimental.pallas import tpu_sc as plsc

B, C, H, W = 8, 768, 32, 32
HW = H * W
NPIX = B * HW
NREF = 4
BLK = 512
LANES = 16
NC, NS = 2, 16
NW = NC * NS
PPW = NPIX // NW
GROUP = 8
GPW = PPW // GROUP
NCHUNK = C // LANES
SCALE = 1.0 / float(C) ** 0.5


def _tc_body(x_ref, wq_ref, wk_ref, wv_ref, wo_ref, bq_ref, bk_ref, bv_ref,
             bo_ref, q_ref, kv_ref, gidx_ref):
    i = pl.program_id(0)
    xb = x_ref[...]
    q = jnp.dot(xb, wq_ref[...], preferred_element_type=jnp.float32) + bq_ref[...]
    k = jnp.dot(xb, wk_ref[...], preferred_element_type=jnp.float32) + bk_ref[...]
    v = jnp.dot(xb, wv_ref[...], preferred_element_type=jnp.float32) + bv_ref[...]
    q_ref[...] = q
    kv_ref[:, :C] = k
    kv_ref[:, C:] = v
    off = jnp.dot(q, wo_ref[...], preferred_element_type=jnp.float32) + bo_ref[...]
    p = i * BLK + lax.broadcasted_iota(jnp.int32, (BLK, 1), 0)
    b = p // HW
    rem = p % HW
    ypix = (rem // W).astype(jnp.float32)
    xpix = (rem % W).astype(jnp.float32)
    cols = []
    for r in range(NREF):
        rx = jnp.floor(jnp.clip(xpix + off[:, 2 * r:2 * r + 1], 0.0, W - 1.0))
        ry = jnp.floor(jnp.clip(ypix + off[:, 2 * r + 1:2 * r + 2], 0.0, H - 1.0))
        cols.append(b * HW + ry.astype(jnp.int32) * W + rx.astype(jnp.int32))
    gidx_ref[...] = jnp.concatenate(cols, axis=1)


_tc_call = pl.pallas_call(
    _tc_body,
    grid=(NPIX // BLK,),
    in_specs=[
        pl.BlockSpec((BLK, C), lambda i: (i, 0)),
        pl.BlockSpec((C, C), lambda i: (0, 0)),
        pl.BlockSpec((C, C), lambda i: (0, 0)),
        pl.BlockSpec((C, C), lambda i: (0, 0)),
        pl.BlockSpec((C, 2 * NREF), lambda i: (0, 0)),
        pl.BlockSpec((1, C), lambda i: (0, 0)),
        pl.BlockSpec((1, C), lambda i: (0, 0)),
        pl.BlockSpec((1, C), lambda i: (0, 0)),
        pl.BlockSpec((1, 2 * NREF), lambda i: (0, 0)),
    ],
    out_specs=[
        pl.BlockSpec((BLK, C), lambda i: (i, 0)),
        pl.BlockSpec((BLK, 2 * C), lambda i: (i, 0)),
        pl.BlockSpec((BLK, NREF), lambda i: (i, 0)),
    ],
    out_shape=[
        jax.ShapeDtypeStruct((NPIX, C), jnp.float32),
        jax.ShapeDtypeStruct((NPIX, 2 * C), jnp.float32),
        jax.ShapeDtypeStruct((NPIX, NREF), jnp.int32),
    ],
)


def _sc_body(q2, kv2, gidxf, out2, idx_v, kvrows, q_v, out_v, sem):
    wid = lax.axis_index("s") * NC + lax.axis_index("c")

    def group(g, _):
        grp = wid * GPW + g
        base = grp * GROUP
        pltpu.sync_copy(gidxf.at[pl.ds(grp * GROUP * NREF, GROUP * NREF)], idx_v)
        cp = pltpu.async_copy(kv2.at[idx_v], kvrows, sem)
        pltpu.sync_copy(q2.at[pl.ds(base, GROUP)], q_v)
        cp.wait()

        def pixel(p, _):
            j0 = p * NREF
            acc = [jnp.zeros((LANES,), jnp.float32) for _ in range(NREF)]
            for cc in range(NCHUNK):
                sl = pl.ds(cc * LANES, LANES)
                qc = q_v[p, sl]
                for r in range(NREF):
                    acc[r] = acc[r] + qc * kvrows[j0 + r, sl]
            att = []
            for r in range(NREF):
                zv = acc[r]
                for sh in (8, 4, 2, 1):
                    perm = (lax.iota(jnp.int32, LANES) + sh) & (LANES - 1)
                    rot = lax.gather(
                        zv, perm[:, None],
                        lax.GatherDimensionNumbers(
                            offset_dims=(), collapsed_slice_dims=(0,),
                            start_index_map=(0,)),
                        slice_sizes=(1,),
                        mode=lax.GatherScatterMode.PROMISE_IN_BOUNDS)
                    zv = zv + rot
                zv = zv * SCALE
                att.append(1.0 / (1.0 + jnp.exp(-zv)))
            for cc in range(NCHUNK):
                slv = pl.ds(C + cc * LANES, LANES)
                o = att[0] * kvrows[j0, slv]
                for r in range(1, NREF):
                    o = o + att[r] * kvrows[j0 + r, slv]
                out_v[p, pl.ds(cc * LANES, LANES)] = o
            return 0

        lax.fori_loop(0, GROUP, pixel, 0)
        pltpu.sync_copy(out_v, out2.at[pl.ds(base, GROUP)])
        return 0

    lax.fori_loop(0, GPW, group, 0)


@functools.cache
def _sc_call():
    return pl.kernel(
        _sc_body,
        out_type=jax.ShapeDtypeStruct((NPIX, C), jnp.float32),
        mesh=plsc.VectorSubcoreMesh(core_axis_name="c", subcore_axis_name="s"),
        scratch_types=[
            pltpu.VMEM((GROUP * NREF,), jnp.int32),
            pltpu.VMEM((GROUP * NREF, 2 * C), jnp.float32),
            pltpu.VMEM((GROUP, C), jnp.float32),
            pltpu.VMEM((GROUP, C), jnp.float32),
            pltpu.SemaphoreType.DMA,
        ],
    )


def kernel(x, Wq, bq, Wk, bk, Wv, bv, Wo, bo):
    x2 = x.reshape(B, C, HW).transpose(0, 2, 1).reshape(NPIX, C)
    q2, kv2, gidx = _tc_call(x2, Wq.T, Wk.T, Wv.T, Wo.T, bq[None, :],
                             bk[None, :], bv[None, :], bo[None, :])
    out2 = _sc_call()(q2, kv2, gidx.reshape(NPIX * NREF))
    return out2.reshape(B, HW, C).transpose(0, 2, 1).reshape(B, C, H, W)


# speedup vs baseline: 12.8357x; 1.3101x over previous
import functools

import jax
import jax.numpy as jnp
from jax import lax
from jax.experimental import pallas as pl
from jax.experimental.pallas import tpu as pltpu
from jax.experimental.pallas import tpu_sc as plsc

B, C, H, W = 8, 768, 32, 32
HW = H * W
NPIX = B * HW
NREF = 4
LANES = 16
NC, NS = 2, 16
NW = NC * NS
PPW = NPIX // NW
GROUP = 8
GPW = PPW // GROUP
NCHUNK = C // LANES
SCALE = 1.0 / float(C) ** 0.5


def _tc_body(x_ref, wq_ref, wk_ref, wv_ref, wo_ref, bq_ref, bk_ref, bv_ref,
             bo_ref, v_ref, s_ref, gidx_ref):
    b = pl.program_id(0)
    xb = x_ref[...]
    q = jnp.dot(xb, wq_ref[...], preferred_element_type=jnp.float32) + bq_ref[...]
    k = jnp.dot(xb, wk_ref[...], preferred_element_type=jnp.float32) + bk_ref[...]
    v_ref[...] = jnp.dot(xb, wv_ref[...], preferred_element_type=jnp.float32) + bv_ref[...]
    s_ref[...] = lax.dot_general(q, k, (((1,), (1,)), ((), ())),
                                 preferred_element_type=jnp.float32)
    off = jnp.dot(q, wo_ref[...], preferred_element_type=jnp.float32) + bo_ref[...]
    p = lax.broadcasted_iota(jnp.int32, (HW, 1), 0)
    ypix = (p // W).astype(jnp.float32)
    xpix = (p % W).astype(jnp.float32)
    cols = []
    for r in range(NREF):
        rx = jnp.floor(jnp.clip(xpix + off[:, 2 * r:2 * r + 1], 0.0, W - 1.0))
        ry = jnp.floor(jnp.clip(ypix + off[:, 2 * r + 1:2 * r + 2], 0.0, H - 1.0))
        cols.append(b * HW + ry.astype(jnp.int32) * W + rx.astype(jnp.int32))
    gidx_ref[...] = jnp.concatenate(cols, axis=1)


_tc_call = pl.pallas_call(
    _tc_body,
    grid=(B,),
    in_specs=[
        pl.BlockSpec((HW, C), lambda i: (i, 0)),
        pl.BlockSpec((C, C), lambda i: (0, 0)),
        pl.BlockSpec((C, C), lambda i: (0, 0)),
        pl.BlockSpec((C, C), lambda i: (0, 0)),
        pl.BlockSpec((C, 2 * NREF), lambda i: (0, 0)),
        pl.BlockSpec((1, C), lambda i: (0, 0)),
        pl.BlockSpec((1, C), lambda i: (0, 0)),
        pl.BlockSpec((1, C), lambda i: (0, 0)),
        pl.BlockSpec((1, 2 * NREF), lambda i: (0, 0)),
    ],
    out_specs=[
        pl.BlockSpec((HW, C), lambda i: (i, 0)),
        pl.BlockSpec((HW, HW), lambda i: (i, 0)),
        pl.BlockSpec((HW, NREF), lambda i: (i, 0)),
    ],
    out_shape=[
        jax.ShapeDtypeStruct((NPIX, C), jnp.float32),
        jax.ShapeDtypeStruct((NPIX, HW), jnp.float32),
        jax.ShapeDtypeStruct((NPIX, NREF), jnp.int32),
    ],
)


def _lane_splat(vec, lane):
    perm = jnp.broadcast_to(lane, (LANES,))
    return lax.gather(
        vec, perm[:, None],
        lax.GatherDimensionNumbers(offset_dims=(), collapsed_slice_dims=(0,),
                                   start_index_map=(0,)),
        slice_sizes=(1,), mode=lax.GatherScatterMode.PROMISE_IN_BOUNDS)


def _sc_body(v2, s2, gidxf, out2, idx_v, vrows, s_v, out_v, sem, sem_s):
    wid = lax.axis_index("s") * NC + lax.axis_index("c")

    def group(g, _):
        grp = wid * GPW + g
        base = grp * GROUP
        pltpu.sync_copy(gidxf.at[pl.ds(grp * GROUP * NREF, GROUP * NREF)], idx_v)
        cp = pltpu.async_copy(v2.at[idx_v], vrows, sem)
        scps = [pltpu.async_copy(s2.at[base + p], s_v.at[pl.ds(p * HW, HW)], sem_s)
                for p in range(GROUP)]
        for scp in scps:
            scp.wait()
        chunks = [idx_v[pl.ds(c * LANES, LANES)]
                  for c in range(GROUP * NREF // LANES)]
        cp.wait()
        for p in range(GROUP):
            avs = []
            for r in range(NREF):
                j = p * NREF + r
                li = chunks[j // LANES][j % LANES] & (HW - 1)
                cvec = s_v[pl.ds(p * HW + (li & ~(LANES - 1)), LANES)]
                zv = _lane_splat(cvec, li & (LANES - 1)) * SCALE
                avs.append(1.0 / (1.0 + jnp.exp(-zv)))
            j0 = p * NREF
            for cc in range(NCHUNK):
                sl = pl.ds(cc * LANES, LANES)
                o = avs[0] * vrows[j0, sl]
                for r in range(1, NREF):
                    o = o + avs[r] * vrows[j0 + r, sl]
                out_v[p, sl] = o
        pltpu.sync_copy(out_v, out2.at[pl.ds(base, GROUP)])
        return 0

    lax.fori_loop(0, GPW, group, 0)


@functools.cache
def _sc_call():
    return pl.kernel(
        _sc_body,
        out_type=jax.ShapeDtypeStruct((NPIX, C), jnp.float32),
        mesh=plsc.VectorSubcoreMesh(core_axis_name="c", subcore_axis_name="s"),
        scratch_types=[
            pltpu.VMEM((GROUP * NREF,), jnp.int32),
            pltpu.VMEM((GROUP * NREF, C), jnp.float32),
            pltpu.VMEM((GROUP * HW,), jnp.float32),
            pltpu.VMEM((GROUP, C), jnp.float32),
            pltpu.SemaphoreType.DMA,
            pltpu.SemaphoreType.DMA,
        ],
    )


def kernel(x, Wq, bq, Wk, bk, Wv, bv, Wo, bo):
    x2 = x.reshape(B, C, HW).transpose(0, 2, 1).reshape(NPIX, C)
    v2, s2, gidx = _tc_call(x2, Wq.T, Wk.T, Wv.T, Wo.T, bq[None, :],
                            bk[None, :], bv[None, :], bo[None, :])
    out2 = _sc_call()(v2, s2, gidx.reshape(NPIX * NREF))
    return out2.reshape(B, HW, C).transpose(0, 2, 1).reshape(B, C, H, W)
